# baseline (speedup 1.0000x reference)
import dataclasses
import functools

import jax
import jax.numpy as jnp
from jax import lax
from jax.experimental import pallas as pl
from jax.experimental.pallas import tpu as pltpu
from jax.experimental.pallas import tpu_sc as plsc

_G = 16
_N = 10000
_E = 320000
_H = 128
_PW = 64
_ROW = 80
_NC = 2
_NS = 16
_NW = _NS
_EW = _E // _NW
_CHUNK = 160
_NCHUNK = _EW // _CHUNK
_NP = 10240
_RPT = _NP // _NS
_BLK = 1000
_NBLK = _N // _BLK


def _enc_body(x_ref, w_ref, b_ref, o_ref):
    o_ref[...] = jnp.dot(x_ref[...], w_ref[...],
                         preferred_element_type=jnp.float32) + b_ref[...]


def _encode(x, W, b):
    return pl.pallas_call(
        _enc_body,
        grid=(_NBLK,),
        in_specs=[pl.BlockSpec((_BLK, _H), lambda i: (i, 0)),
                  pl.BlockSpec((_H, _H), lambda i: (0, 0)),
                  pl.BlockSpec((1, _H), lambda i: (0, 0))],
        out_specs=pl.BlockSpec((_BLK, _H), lambda i: (i, 0)),
        out_shape=jax.ShapeDtypeStruct((_N, _H), jnp.float32),
    )(x, W, b.reshape(1, _H))


def _tca_body(h_ref, w_ref, a8_ref, m_ref, ht2_ref, s_ref, ssum_ref):
    i = pl.program_id(0)
    ht = jnp.dot(h_ref[...], w_ref[...], preferred_element_type=jnp.float32)
    ht2_ref[0] = ht[:, :_PW]
    ht2_ref[1] = ht[:, _PW:]
    s8 = lax.dot_general(a8_ref[...], ht, (((1,), (1,)), ((), ())),
                         preferred_element_type=jnp.float32)
    s_ref[0] = s8
    ssum_ref[...] = jnp.dot(ht, m_ref[...],
                            preferred_element_type=jnp.float32)


def _tca(h, W, a8, M):
    return pl.pallas_call(
        _tca_body,
        grid=(_NBLK,),
        in_specs=[pl.BlockSpec((_BLK, _H), lambda i: (i, 0)),
                  pl.BlockSpec((_H, _H), lambda i: (0, 0)),
                  pl.BlockSpec((8, _H), lambda i: (0, 0)),
                  pl.BlockSpec((_H, _H), lambda i: (0, 0))],
        out_specs=[pl.BlockSpec((_NC, _BLK, _PW), lambda i: (0, i, 0)),
                   pl.BlockSpec((1, 8, _BLK), lambda i: (i, 0, 0)),
                   pl.BlockSpec((_BLK, _H), lambda i: (i, 0))],
        out_shape=[jax.ShapeDtypeStruct((_NC, _N, _PW), jnp.float32),
                   jax.ShapeDtypeStruct((_NBLK, 8, _BLK), jnp.float32),
                   jax.ShapeDtypeStruct((_N, _H), jnp.float32)],
    )(h, W, a8, M)


def _tcc_body(p0_ref, p1_ref, ht2_ref, ssum_ref, b_ref, o_ref):
    p0 = p0_ref[...]
    p1 = p1_ref[...]
    numer = jnp.concatenate([p0[:, :_PW], p1[:, :_PW]], axis=1)
    ht = jnp.concatenate([ht2_ref[0], ht2_ref[1]], axis=1)
    den = p0[:, _PW:_PW + 1]
    eas = p0[:, _PW + 1:_PW + 2]
    cn = p0[:, _PW + 2:_PW + 3]
    a = ssum_ref[...] + eas / jnp.maximum(cn, 1.0)
    wself = jnp.exp(jnp.maximum(a, 0.2 * a))
    o_ref[...] = jnp.maximum(
        (numer + wself * ht) / (den + wself) + b_ref[...], 0.0)


def _tcc(p0, p1, ht2, ssum, b):
    return pl.pallas_call(
        _tcc_body,
        grid=(_NBLK,),
        in_specs=[pl.BlockSpec((_BLK, _ROW), lambda i: (i, 0)),
                  pl.BlockSpec((_BLK, _ROW), lambda i: (i, 0)),
                  pl.BlockSpec((_NC, _BLK, _PW), lambda i: (0, i, 0)),
                  pl.BlockSpec((_BLK, _H), lambda i: (i, 0)),
                  pl.BlockSpec((1, _H), lambda i: (0, 0))],
        out_specs=pl.BlockSpec((_BLK, _H), lambda i: (i, 0)),
        out_shape=jax.ShapeDtypeStruct((_N, _H), jnp.float32),
    )(p0, p1, ht2, ssum, b.reshape(1, _H))


def _tcp_body(h_ref, b3_ref, o_ref, pool, cntv):
    i = pl.program_id(0)

    @pl.when(i == 0)
    def _():
        pool[...] = jnp.zeros_like(pool)
        cntv[...] = jnp.zeros_like(cntv)

    b2 = b3_ref[0]
    gids = lax.broadcasted_iota(jnp.int32, (_G, _BLK), 0)
    mask = (jnp.broadcast_to(b2, (_G, _BLK)) == gids).astype(jnp.float32)
    pool[...] += jnp.dot(mask, h_ref[...], preferred_element_type=jnp.float32)
    cntv[...] += jnp.sum(mask, axis=1, keepdims=True) \
        * jnp.ones((1, _H), jnp.float32)

    @pl.when(i == pl.num_programs(0) - 1)
    def _():
        o_ref[...] = pool[...] / jnp.maximum(cntv[...], 1.0)


def _tcp(h, batch3):
    return pl.pallas_call(
        _tcp_body,
        grid=(_NBLK,),
        in_specs=[pl.BlockSpec((_BLK, _H), lambda i: (i, 0)),
                  pl.BlockSpec((1, 1, _BLK), lambda i: (i, 0, 0))],
        out_specs=pl.BlockSpec((_G, _H), lambda i: (0, 0)),
        out_shape=jax.ShapeDtypeStruct((_G, _H), jnp.float32),
        scratch_shapes=[pltpu.VMEM((_G, _H), jnp.float32),
                        pltpu.VMEM((_G, _H), jnp.float32)],
    )(h, batch3)


def _sc_edge_body(ht_hbm, s_hbm, pack_hbm, vv_hbm, out_hbm,
                  ssrc_v, sdst_v, pack_v, vv_v, gbuf, stage,
                  acc, sem, sem2, sem3):
    cid = lax.axis_index("c")
    sid = lax.axis_index("s")

    score_cps = []
    for i in range(_NBLK):
        score_cps.append(pltpu.async_copy(
            s_hbm.at[i, 0], ssrc_v.at[pl.ds(i * _BLK, _BLK)], sem2))
        score_cps.append(pltpu.async_copy(
            s_hbm.at[i, 1], sdst_v.at[pl.ds(i * _BLK, _BLK)], sem2))
    pltpu.sync_copy(vv_hbm, vv_v)
    for cp in score_cps:
        cp.wait()

    z16 = jnp.zeros((16,), jnp.float32)

    @pl.loop(0, 2)
    def _(b):
        @pl.loop(0, _CHUNK)
        def _(r):
            @pl.loop(0, _ROW // 16)
            def _(j):
                stage[b, r, pl.ds(j * 16, 16)] = z16

    @pl.loop(0, _RPT // _CHUNK)
    def _(k):
        pltpu.sync_copy(stage.at[0],
                        acc.at[pl.ds(sid * _RPT + k * _CHUNK, _CHUNK)])

    plsc.subcore_barrier()

    iota16 = lax.iota(jnp.int32, 16)
    vv0 = vv_v[0, :]
    vv1 = vv_v[1, :]
    vv2 = vv_v[2, :]
    vv3 = vv_v[3, :]
    ones16 = jnp.ones((16,), jnp.float32)
    cw = jnp.full((16,), _PW, jnp.int32)
    cea = jnp.full((16,), _PW + 1, jnp.int32)
    cone = jnp.full((16,), _PW + 2, jnp.int32)
    my_ht = ht_hbm.at[cid]

    bf32 = functools.partial(lax.bitcast_convert_type,
                             new_dtype=jnp.float32)
    pltpu.sync_copy(pack_hbm.at[sid, 0], pack_v.at[0])
    pltpu.async_copy(pack_hbm.at[sid, 1], pack_v.at[1], sem2)
    pltpu.async_copy(my_ht.at[pack_v.at[0, 0]], gbuf.at[0], sem)

    @pl.loop(0, _NCHUNK)
    def _(g):
        s0 = lax.rem(g, 5)
        s1 = lax.rem(g + 1, 5)
        s2 = lax.rem(g + 2, 5)
        cb = lax.rem(g, 2)
        nb = lax.rem(g + 1, 2)
        cbs = jnp.broadcast_to(cb, (16,))

        @pl.when(g + 1 < _NCHUNK)
        def _():
            pltpu.make_async_copy(pack_hbm.at[sid, 0], pack_v.at[0],
                                  sem2).wait()
            pltpu.async_copy(my_ht.at[pack_v.at[s1, 0]], gbuf.at[nb], sem)

        @pl.when(g + 2 < _NCHUNK)
        def _():
            pltpu.async_copy(pack_hbm.at[sid, g + 2], pack_v.at[s2], sem2)

        @pl.when(g >= 2)
        def _():
            pltpu.make_async_copy(stage.at[cb], acc.at[pack_v.at[0, 1]],
                                  sem3).wait()

        @plsc.parallel_loop(0, _CHUNK // 16, unroll=2)
        def _(j):
            s16 = pack_v[s0, 0, pl.ds(j * 16, 16)]
            d16 = pack_v[s0, 1, pl.ds(j * 16, 16)]
            ea16 = (bf32(pack_v[s0, 2, pl.ds(j * 16, 16)]) * vv0
                    + bf32(pack_v[s0, 3, pl.ds(j * 16, 16)]) * vv1
                    + bf32(pack_v[s0, 4, pl.ds(j * 16, 16)]) * vv2
                    + bf32(pack_v[s0, 5, pl.ds(j * 16, 16)]) * vv3)
            a16 = (plsc.load_gather(ssrc_v, [s16])
                   + plsc.load_gather(sdst_v, [d16]) + ea16)
            w16 = jnp.exp(jnp.maximum(a16, a16 * 0.2))
            ridx = j * 16 + iota16
            plsc.store_scatter(stage, [cbs, ridx, cw], w16)
            plsc.store_scatter(stage, [cbs, ridx, cea], ea16)
            plsc.store_scatter(stage, [cbs, ridx, cone], ones16)

        pltpu.make_async_copy(my_ht.at[pack_v.at[s0, 0]], gbuf.at[cb],
                              sem).wait()

        @plsc.parallel_loop(0, _CHUNK, unroll=8)
        def _(r):
            rr = jnp.broadcast_to(r, (16,))
            w16 = plsc.load_gather(stage, [cbs, rr, cw])
            for j in range(_PW // 16):
                stage[cb, r, pl.ds(j * 16, 16)] = \
                    gbuf[cb, r, pl.ds(j * 16, 16)] * w16

        pltpu.async_copy(stage.at[cb], acc.at[pack_v.at[s0, 1]], sem3,
                         add=True)

    pltpu.make_async_copy(stage.at[0], acc.at[pack_v.at[0, 1]], sem3).wait()
    pltpu.make_async_copy(stage.at[1], acc.at[pack_v.at[0, 1]], sem3).wait()

    plsc.subcore_barrier()

    pltpu.sync_copy(acc.at[pl.ds(sid * _RPT, _RPT)],
                    out_hbm.at[cid, pl.ds(sid * _RPT, _RPT)])


def _sc_compiler_params():
    cp = pltpu.CompilerParams()
    fields = pltpu.CompilerParams.__dataclass_fields__
    if "needs_layout_passes" in fields:
        cp = dataclasses.replace(cp, needs_layout_passes=False)
    if "use_tc_tiling_on_sc" in fields:
        cp = dataclasses.replace(cp, use_tc_tiling_on_sc=False)
    return cp


def _sc_edge(ht2, S, pack, vv):
    kern = pl.kernel(
        _sc_edge_body,
        out_type=jax.ShapeDtypeStruct((_NC, _NP, _ROW), jnp.float32),
        mesh=plsc.VectorSubcoreMesh(core_axis_name="c", subcore_axis_name="s"),
        scratch_types=[
            pltpu.VMEM((_N,), jnp.float32),
            pltpu.VMEM((_N,), jnp.float32),
            pltpu.VMEM((5, 6, _CHUNK), jnp.int32),
            pltpu.VMEM((4, 16), jnp.float32),
            pltpu.VMEM((2, _CHUNK, _PW), jnp.float32),
            pltpu.VMEM((2, _CHUNK, _ROW), jnp.float32),
            pltpu.VMEM_SHARED((_NP, _ROW), jnp.float32),
            pltpu.SemaphoreType.DMA,
            pltpu.SemaphoreType.DMA,
            pltpu.SemaphoreType.DMA,
        ],
        compiler_params=_sc_compiler_params(),
    )
    return kern(ht2, S, pack, vv)


def kernel(x, edge_index, edge_attr, batch, W_enc, b_enc, W_lin, att_src,
           att_dst, W_edge, att_edge, b_gat):
    src = edge_index[0].astype(jnp.int32)
    dst = edge_index[1].astype(jnp.int32)
    nl = W_lin.shape[0]

    src_r = src.reshape(_NW, _NCHUNK, 1, _CHUNK)
    dst_r = dst.reshape(_NW, _NCHUNK, 1, _CHUNK)
    eab = lax.bitcast_convert_type(edge_attr, jnp.int32)
    eat_r = eab.T.reshape(4, _NW, _NCHUNK, _CHUNK).transpose(1, 2, 0, 3)
    pack = jnp.concatenate([src_r, dst_r, eat_r], axis=2)
    batch3 = batch.astype(jnp.int32).reshape(_NBLK, 1, _BLK)
    V = jnp.einsum("lch,lh->lc", W_edge, att_edge)

    h = _encode(x, W_enc, b_enc)
    for l in range(nl):
        a8 = jnp.concatenate(
            [att_src[l][None, :], att_dst[l][None, :],
             jnp.zeros((6, _H), jnp.float32)], axis=0)
        M = jnp.broadcast_to((att_src[l] + att_dst[l])[:, None], (_H, _H))
        ht2, S, ssum = _tca(h, W_lin[l], a8, M)
        vv = jnp.broadcast_to(V[l][:, None], (4, 16))
        P = _sc_edge(ht2, S, pack, vv)
        h = _tcc(P[0], P[1], ht2, ssum, b_gat[l])
    return _tcp(h, batch3)


# speedup vs baseline: 1.0214x; 1.0214x over previous
import dataclasses
import functools

import jax
import jax.numpy as jnp
from jax import lax
from jax.experimental import pallas as pl
from jax.experimental.pallas import tpu as pltpu
from jax.experimental.pallas import tpu_sc as plsc

_G = 16
_N = 10000
_E = 320000
_H = 128
_PW = 64
_ROW = 80
_NC = 2
_NS = 16
_NW = _NS
_EW = _E // _NW
_CHUNK = 160
_NCHUNK = _EW // _CHUNK
_NP = 10240
_RPT = _NP // _NS
_BLK = 1000
_NBLK = _N // _BLK


def _enc_body(x_ref, w_ref, b_ref, o_ref):
    o_ref[...] = jnp.dot(x_ref[...], w_ref[...],
                         preferred_element_type=jnp.float32) + b_ref[...]


def _encode(x, W, b):
    return pl.pallas_call(
        _enc_body,
        grid=(_NBLK,),
        in_specs=[pl.BlockSpec((_BLK, _H), lambda i: (i, 0)),
                  pl.BlockSpec((_H, _H), lambda i: (0, 0)),
                  pl.BlockSpec((1, _H), lambda i: (0, 0))],
        out_specs=pl.BlockSpec((_BLK, _H), lambda i: (i, 0)),
        out_shape=jax.ShapeDtypeStruct((_N, _H), jnp.float32),
    )(x, W, b.reshape(1, _H))


def _tca_body(h_ref, w_ref, a8_ref, m_ref, ht2_ref, s_ref, ssum_ref):
    i = pl.program_id(0)
    ht = jnp.dot(h_ref[...], w_ref[...], preferred_element_type=jnp.float32)
    ht2_ref[0] = ht[:, :_PW]
    ht2_ref[1] = ht[:, _PW:]
    s8 = lax.dot_general(a8_ref[...], ht, (((1,), (1,)), ((), ())),
                         preferred_element_type=jnp.float32)
    s_ref[0] = s8
    ssum_ref[...] = jnp.dot(ht, m_ref[...],
                            preferred_element_type=jnp.float32)


def _tca(h, W, a8, M):
    return pl.pallas_call(
        _tca_body,
        grid=(_NBLK,),
        in_specs=[pl.BlockSpec((_BLK, _H), lambda i: (i, 0)),
                  pl.BlockSpec((_H, _H), lambda i: (0, 0)),
                  pl.BlockSpec((8, _H), lambda i: (0, 0)),
                  pl.BlockSpec((_H, _H), lambda i: (0, 0))],
        out_specs=[pl.BlockSpec((_NC, _BLK, _PW), lambda i: (0, i, 0)),
                   pl.BlockSpec((1, 8, _BLK), lambda i: (i, 0, 0)),
                   pl.BlockSpec((_BLK, _H), lambda i: (i, 0))],
        out_shape=[jax.ShapeDtypeStruct((_NC, _N, _PW), jnp.float32),
                   jax.ShapeDtypeStruct((_NBLK, 8, _BLK), jnp.float32),
                   jax.ShapeDtypeStruct((_N, _H), jnp.float32)],
    )(h, W, a8, M)


def _tcc_body(p0_ref, p1_ref, ht2_ref, ssum_ref, b_ref, o_ref):
    p0 = p0_ref[...]
    p1 = p1_ref[...]
    numer = jnp.concatenate([p0[:, :_PW], p1[:, :_PW]], axis=1)
    ht = jnp.concatenate([ht2_ref[0], ht2_ref[1]], axis=1)
    den = p0[:, _PW:_PW + 1]
    eas = p0[:, _PW + 1:_PW + 2]
    cn = p0[:, _PW + 2:_PW + 3]
    a = ssum_ref[...] + eas / jnp.maximum(cn, 1.0)
    wself = jnp.exp(jnp.maximum(a, 0.2 * a))
    o_ref[...] = jnp.maximum(
        (numer + wself * ht) / (den + wself) + b_ref[...], 0.0)


def _tcc(p0, p1, ht2, ssum, b):
    return pl.pallas_call(
        _tcc_body,
        grid=(_NBLK,),
        in_specs=[pl.BlockSpec((_BLK, _ROW), lambda i: (i, 0)),
                  pl.BlockSpec((_BLK, _ROW), lambda i: (i, 0)),
                  pl.BlockSpec((_NC, _BLK, _PW), lambda i: (0, i, 0)),
                  pl.BlockSpec((_BLK, _H), lambda i: (i, 0)),
                  pl.BlockSpec((1, _H), lambda i: (0, 0))],
        out_specs=pl.BlockSpec((_BLK, _H), lambda i: (i, 0)),
        out_shape=jax.ShapeDtypeStruct((_N, _H), jnp.float32),
    )(p0, p1, ht2, ssum, b.reshape(1, _H))


def _tcp_body(h_ref, b3_ref, o_ref, pool, cntv):
    i = pl.program_id(0)

    @pl.when(i == 0)
    def _():
        pool[...] = jnp.zeros_like(pool)
        cntv[...] = jnp.zeros_like(cntv)

    b2 = b3_ref[0]
    gids = lax.broadcasted_iota(jnp.int32, (_G, _BLK), 0)
    mask = (jnp.broadcast_to(b2, (_G, _BLK)) == gids).astype(jnp.float32)
    pool[...] += jnp.dot(mask, h_ref[...], preferred_element_type=jnp.float32)
    cntv[...] += jnp.sum(mask, axis=1, keepdims=True) \
        * jnp.ones((1, _H), jnp.float32)

    @pl.when(i == pl.num_programs(0) - 1)
    def _():
        o_ref[...] = pool[...] / jnp.maximum(cntv[...], 1.0)


def _tcp(h, batch3):
    return pl.pallas_call(
        _tcp_body,
        grid=(_NBLK,),
        in_specs=[pl.BlockSpec((_BLK, _H), lambda i: (i, 0)),
                  pl.BlockSpec((1, 1, _BLK), lambda i: (i, 0, 0))],
        out_specs=pl.BlockSpec((_G, _H), lambda i: (0, 0)),
        out_shape=jax.ShapeDtypeStruct((_G, _H), jnp.float32),
        scratch_shapes=[pltpu.VMEM((_G, _H), jnp.float32),
                        pltpu.VMEM((_G, _H), jnp.float32)],
    )(h, batch3)


def _sc_edge_body(ht_hbm, s_hbm, pack_hbm, vv_hbm, out_hbm,
                  ssrc_v, sdst_v, pack_v, vv_v, gbuf, stage,
                  acc, sem, sem2, sem3):
    cid = lax.axis_index("c")
    sid = lax.axis_index("s")

    score_cps = []
    for i in range(_NBLK):
        score_cps.append(pltpu.async_copy(
            s_hbm.at[i, 0], ssrc_v.at[pl.ds(i * _BLK, _BLK)], sem2))
        score_cps.append(pltpu.async_copy(
            s_hbm.at[i, 1], sdst_v.at[pl.ds(i * _BLK, _BLK)], sem2))
    pltpu.sync_copy(vv_hbm, vv_v)
    for cp in score_cps:
        cp.wait()

    z16 = jnp.zeros((16,), jnp.float32)

    @pl.loop(0, 2)
    def _(b):
        @pl.loop(0, _CHUNK)
        def _(r):
            @pl.loop(0, _ROW // 16)
            def _(j):
                stage[b, r, pl.ds(j * 16, 16)] = z16

    @pl.loop(0, _RPT // _CHUNK)
    def _(k):
        pltpu.sync_copy(stage.at[0],
                        acc.at[pl.ds(sid * _RPT + k * _CHUNK, _CHUNK)])

    plsc.subcore_barrier()

    iota16 = lax.iota(jnp.int32, 16)
    vv0 = vv_v[0, :]
    vv1 = vv_v[1, :]
    vv2 = vv_v[2, :]
    vv3 = vv_v[3, :]
    ones16 = jnp.ones((16,), jnp.float32)
    cw = jnp.full((16,), _PW, jnp.int32)
    cea = jnp.full((16,), _PW + 1, jnp.int32)
    cone = jnp.full((16,), _PW + 2, jnp.int32)
    my_ht = ht_hbm.at[cid]

    bf32 = functools.partial(lax.bitcast_convert_type,
                             new_dtype=jnp.float32)
    pltpu.sync_copy(pack_hbm.at[sid, 0], pack_v.at[0])
    pltpu.async_copy(pack_hbm.at[sid, 1], pack_v.at[1], sem2)
    pltpu.async_copy(my_ht.at[pack_v.at[0, 0]], gbuf.at[0], sem)

    @pl.loop(0, _NCHUNK)
    def _(g):
        s0 = lax.rem(g, 5)
        s1 = lax.rem(g + 1, 5)
        s2 = lax.rem(g + 2, 5)
        cb = lax.rem(g, 2)
        nb = lax.rem(g + 1, 2)
        cbs = jnp.broadcast_to(cb, (16,))

        @pl.when(g + 1 < _NCHUNK)
        def _():
            pltpu.make_async_copy(pack_hbm.at[sid, 0], pack_v.at[0],
                                  sem2).wait()
            pltpu.async_copy(my_ht.at[pack_v.at[s1, 0]], gbuf.at[nb], sem)

        @pl.when(g + 2 < _NCHUNK)
        def _():
            pltpu.async_copy(pack_hbm.at[sid, g + 2], pack_v.at[s2], sem2)

        @pl.when(g >= 2)
        def _():
            pltpu.make_async_copy(stage.at[cb], acc.at[pack_v.at[0, 1]],
                                  sem3).wait()

        @pl.loop(0, _CHUNK // 16)
        def _(j):
            s16 = pack_v[s0, 0, pl.ds(j * 16, 16)]
            d16 = pack_v[s0, 1, pl.ds(j * 16, 16)]
            ea16 = (bf32(pack_v[s0, 2, pl.ds(j * 16, 16)]) * vv0
                    + bf32(pack_v[s0, 3, pl.ds(j * 16, 16)]) * vv1
                    + bf32(pack_v[s0, 4, pl.ds(j * 16, 16)]) * vv2
                    + bf32(pack_v[s0, 5, pl.ds(j * 16, 16)]) * vv3)
            a16 = (plsc.load_gather(ssrc_v, [s16])
                   + plsc.load_gather(sdst_v, [d16]) + ea16)
            w16 = jnp.exp(jnp.maximum(a16, a16 * 0.2))
            ridx = j * 16 + iota16
            plsc.store_scatter(stage, [cbs, ridx, cw], w16)
            plsc.store_scatter(stage, [cbs, ridx, cea], ea16)
            plsc.store_scatter(stage, [cbs, ridx, cone], ones16)

        pltpu.make_async_copy(my_ht.at[pack_v.at[s0, 0]], gbuf.at[cb],
                              sem).wait()

        @plsc.parallel_loop(0, _CHUNK, unroll=4)
        def _(r):
            rr = jnp.broadcast_to(r, (16,))
            w16 = plsc.load_gather(stage, [cbs, rr, cw])
            for j in range(_PW // 16):
                stage[cb, r, pl.ds(j * 16, 16)] = \
                    gbuf[cb, r, pl.ds(j * 16, 16)] * w16

        pltpu.async_copy(stage.at[cb], acc.at[pack_v.at[s0, 1]], sem3,
                         add=True)

    pltpu.make_async_copy(stage.at[0], acc.at[pack_v.at[0, 1]], sem3).wait()
    pltpu.make_async_copy(stage.at[1], acc.at[pack_v.at[0, 1]], sem3).wait()

    plsc.subcore_barrier()

    pltpu.sync_copy(acc.at[pl.ds(sid * _RPT, _RPT)],
                    out_hbm.at[cid, pl.ds(sid * _RPT, _RPT)])


def _sc_compiler_params():
    cp = pltpu.CompilerParams()
    fields = pltpu.CompilerParams.__dataclass_fields__
    if "needs_layout_passes" in fields:
        cp = dataclasses.replace(cp, needs_layout_passes=False)
    if "use_tc_tiling_on_sc" in fields:
        cp = dataclasses.replace(cp, use_tc_tiling_on_sc=False)
    return cp


def _sc_edge(ht2, S, pack, vv):
    kern = pl.kernel(
        _sc_edge_body,
        out_type=jax.ShapeDtypeStruct((_NC, _NP, _ROW), jnp.float32),
        mesh=plsc.VectorSubcoreMesh(core_axis_name="c", subcore_axis_name="s"),
        scratch_types=[
            pltpu.VMEM((_N,), jnp.float32),
            pltpu.VMEM((_N,), jnp.float32),
            pltpu.VMEM((5, 6, _CHUNK), jnp.int32),
            pltpu.VMEM((4, 16), jnp.float32),
            pltpu.VMEM((2, _CHUNK, _PW), jnp.float32),
            pltpu.VMEM((2, _CHUNK, _ROW), jnp.float32),
            pltpu.VMEM_SHARED((_NP, _ROW), jnp.float32),
            pltpu.SemaphoreType.DMA,
            pltpu.SemaphoreType.DMA,
            pltpu.SemaphoreType.DMA,
        ],
        compiler_params=_sc_compiler_params(),
    )
    return kern(ht2, S, pack, vv)


def kernel(x, edge_index, edge_attr, batch, W_enc, b_enc, W_lin, att_src,
           att_dst, W_edge, att_edge, b_gat):
    src = edge_index[0].astype(jnp.int32)
    dst = edge_index[1].astype(jnp.int32)
    nl = W_lin.shape[0]

    src_r = src.reshape(_NW, _NCHUNK, 1, _CHUNK)
    dst_r = dst.reshape(_NW, _NCHUNK, 1, _CHUNK)
    eab = lax.bitcast_convert_type(edge_attr, jnp.int32)
    eat_r = eab.T.reshape(4, _NW, _NCHUNK, _CHUNK).transpose(1, 2, 0, 3)
    pack = jnp.concatenate([src_r, dst_r, eat_r], axis=2)
    batch3 = batch.astype(jnp.int32).reshape(_NBLK, 1, _BLK)
    V = jnp.einsum("lch,lh->lc", W_edge, att_edge)

    h = _encode(x, W_enc, b_enc)
    for l in range(nl):
        a8 = jnp.concatenate(
            [att_src[l][None, :], att_dst[l][None, :],
             jnp.zeros((6, _H), jnp.float32)], axis=0)
        M = jnp.broadcast_to((att_src[l] + att_dst[l])[:, None], (_H, _H))
        ht2, S, ssum = _tca(h, W_lin[l], a8, M)
        vv = jnp.broadcast_to(V[l][:, None], (4, 16))
        P = _sc_edge(ht2, S, pack, vv)
        h = _tcc(P[0], P[1], ht2, ssum, b_gat[l])
    return _tcp(h, batch3)
